# Initial kernel scaffold; baseline (speedup 1.0000x reference)
import jax
import jax.numpy as jnp
from jax.experimental import pallas as pl


def kernel(x, embed_weight, tocls_weight, tocls_bias):
    raise NotImplementedError("write your pallas kernel here")


# speedup vs baseline: 2.1504x; 2.1504x over previous
import functools

import jax
import jax.numpy as jnp
from jax import lax
from jax.experimental import pallas as pl
from jax.experimental.pallas import tpu as pltpu
from jax.experimental.pallas import tpu_sc as plsc

_K = 32
_CLS = 128
_B = 4096
_L = 200
_NC = 2
_NS = 16
_NW = _NC * _NS
_BPW = _B // _NW
_C0, _C1 = 128, 72
_NJ = _CLS // 16


def _make_sc_kernel(V):
    mesh = plsc.VectorSubcoreMesh(core_axis_name="c", subcore_axis_name="s")

    @functools.partial(
        pl.kernel,
        mesh=mesh,
        out_type=jax.ShapeDtypeStruct((_B, _CLS), jnp.float32),
        compiler_params=pltpu.CompilerParams(use_tc_tiling_on_sc=False),
        scratch_types=[
            pltpu.VMEM((_BPW, _L), jnp.int32),
            pltpu.VMEM((2, _L, _K), jnp.float32),
            pltpu.VMEM((_K, _CLS), jnp.float32),
            pltpu.VMEM((_CLS,), jnp.float32),
            pltpu.VMEM((_BPW, _CLS), jnp.float32),
            pltpu.SemaphoreType.DMA,
            pltpu.SemaphoreType.DMA,
        ],
    )
    def sc_kernel(x_hbm, tab_hbm, wt_hbm, bias_hbm, out_hbm,
                  idx_v, rows_v, wt_v, bias_v, out_v, sem0, sem1):
        wid = lax.axis_index("s") * _NC + lax.axis_index("c")
        base = wid * _BPW
        pltpu.sync_copy(x_hbm.at[pl.ds(base, _BPW)], idx_v)
        pltpu.sync_copy(wt_hbm, wt_v)
        pltpu.sync_copy(bias_hbm, bias_v)

        def issue(b, buf, sem):
            pltpu.async_copy(tab_hbm.at[idx_v.at[b, pl.ds(0, _C0)]],
                             rows_v.at[buf, pl.ds(0, _C0)], sem)
            pltpu.async_copy(tab_hbm.at[idx_v.at[b, pl.ds(_C0, _C1)]],
                             rows_v.at[buf, pl.ds(_C0, _C1)], sem)

        def wait(b, buf, sem):
            pltpu.make_async_copy(tab_hbm.at[idx_v.at[b, pl.ds(0, _C0)]],
                                  rows_v.at[buf, pl.ds(0, _C0)], sem).wait()
            pltpu.make_async_copy(tab_hbm.at[idx_v.at[b, pl.ds(_C0, _C1)]],
                                  rows_v.at[buf, pl.ds(_C0, _C1)], sem).wait()

        def process(b, buf):
            def red(r, acc):
                a0, a1 = acc
                return (a0 + rows_v[buf, r, pl.ds(0, 16)],
                        a1 + rows_v[buf, r, pl.ds(16, 16)])
            zero = jnp.zeros((16,), jnp.float32)
            a0, a1 = lax.fori_loop(0, _L, red, (zero, zero))
            scale = jnp.float32(1.0 / _L)
            m = (a0 * scale, a1 * scale)
            accs = [bias_v[pl.ds(j * 16, 16)] for j in range(_NJ)]
            for k in range(_K):
                s = m[k // 16][k % 16]
                for j in range(_NJ):
                    accs[j] = accs[j] + s * wt_v[k, pl.ds(j * 16, 16)]
            for j in range(_NJ):
                out_v[b, pl.ds(j * 16, 16)] = accs[j]

        issue(0, 0, sem0)

        def loop_body(i, carry):
            b0 = i * 2
            issue(b0 + 1, 1, sem1)
            wait(b0, 0, sem0)
            process(b0, 0)

            @pl.when(b0 + 2 < _BPW)
            def _prefetch():
                issue(b0 + 2, 0, sem0)

            wait(b0 + 1, 1, sem1)
            process(b0 + 1, 1)
            return carry

        lax.fori_loop(0, _BPW // 2, loop_body, 0)
        pltpu.sync_copy(out_v, out_hbm.at[pl.ds(base, _BPW)])

    return sc_kernel


@jax.jit
def kernel(x, embed_weight, tocls_weight, tocls_bias):
    xi = x.astype(jnp.int32)
    wt = jnp.transpose(tocls_weight)
    sc = _make_sc_kernel(embed_weight.shape[0])
    return sc(xi, embed_weight, wt, tocls_bias)
